# Initial kernel scaffold; baseline (speedup 1.0000x reference)
import jax
import jax.numpy as jnp
from jax.experimental import pallas as pl


def kernel(x, edge_index, Wl1, Wr1, b1, Wl2, Wr2, b2, Wl3, Wr3, b3):
    raise NotImplementedError("write your pallas kernel here")


# speedup vs baseline: 5.2463x; 5.2463x over previous
import functools

import jax
import jax.numpy as jnp
from jax import lax
from jax.experimental import pallas as pl
from jax.experimental.pallas import tpu as pltpu
from jax.experimental.pallas import tpu_sc as plsc

N = 10000
E = 320000
D = 128
HD = D // 2

NC = 2
NS = 16

CHUNK = 128
NCHUNK = 160
EPT = CHUNK * NCHUNK
E_PAD = NS * EPT

NB = 3

ACC_ROWS = 10240
ROWS_PER_SUB = ACC_ROWS // NS
PAD_DST = N


def _sc_segsum(with_cnt):
    mesh = plsc.VectorSubcoreMesh(
        core_axis_name="c", subcore_axis_name="s", num_cores=NC, num_subcores=NS)

    out_type = [jax.ShapeDtypeStruct((NC, ACC_ROWS, HD), jnp.float32)]
    scratch = [
        pltpu.VMEM((NCHUNK, CHUNK), jnp.int32),
        pltpu.VMEM((NCHUNK, CHUNK), jnp.int32),
        pltpu.VMEM((CHUNK, HD), jnp.float32),
        pltpu.VMEM_SHARED((ACC_ROWS, HD), jnp.float32),
    ] + [pltpu.VMEM((CHUNK, HD), jnp.float32) for _ in range(NB)] \
      + [pltpu.SemaphoreType.DMA for _ in range(2 * NB)]
    if with_cnt:
        out_type.append(jax.ShapeDtypeStruct((ACC_ROWS, 16), jnp.float32))
        scratch += [
            pltpu.VMEM((CHUNK, 16), jnp.float32),
            pltpu.VMEM((CHUNK, 16), jnp.float32),
            pltpu.VMEM_SHARED((ACC_ROWS, 16), jnp.float32),
        ] + [pltpu.SemaphoreType.DMA for _ in range(NB)]

    def body(y0_hbm, y1_hbm, src_hbm, dst_hbm, zeros_hbm, ones_hbm, zeros16_hbm,
             *refs):
        if with_cnt:
            (parts_out, cnt_out, src_v, dst_v, zbuf, acc, *rest) = refs
            bufs = rest[:NB]
            gsem = rest[NB:2 * NB]
            ssem = rest[2 * NB:3 * NB]
            obuf, zbuf16, cnt_acc = rest[3 * NB:3 * NB + 3]
            csem = rest[3 * NB + 3:]
        else:
            (parts_out, src_v, dst_v, zbuf, acc, *rest) = refs
            bufs = rest[:NB]
            gsem = rest[NB:2 * NB]
            ssem = rest[2 * NB:3 * NB]

        cid = lax.axis_index("c")
        sid = lax.axis_index("s")
        base = sid * ROWS_PER_SUB

        pltpu.sync_copy(src_hbm.at[sid], src_v)
        pltpu.sync_copy(dst_hbm.at[sid], dst_v)
        pltpu.sync_copy(zeros_hbm, zbuf)

        def pipeline(y_hbm, do_cnt):
            pltpu.async_copy(y_hbm.at[src_v.at[0]], bufs[0], gsem[0])
            pltpu.async_copy(y_hbm.at[src_v.at[1]], bufs[1], gsem[1])

            for k in range(ROWS_PER_SUB // CHUNK):
                pltpu.sync_copy(zbuf, acc.at[pl.ds(base + k * CHUNK, CHUNK)])
            if do_cnt:
                pltpu.sync_copy(ones_hbm, obuf)
                pltpu.sync_copy(zeros16_hbm, zbuf16)
                for k in range(ROWS_PER_SUB // CHUNK):
                    pltpu.sync_copy(zbuf16,
                                    cnt_acc.at[pl.ds(base + k * CHUNK, CHUNK)])
            plsc.subcore_barrier()

            def wait_gather(j, b):
                pltpu.make_async_copy(y_hbm.at[src_v.at[j]], bufs[b],
                                      gsem[b]).wait()

            def issue_scatter(j, b):
                pltpu.async_copy(bufs[b], acc.at[dst_v.at[j]], ssem[b],
                                 add=True)
                if do_cnt:
                    pltpu.async_copy(obuf, cnt_acc.at[dst_v.at[j]], csem[b],
                                     add=True)

            def wait_scatter(j, b):
                pltpu.make_async_copy(bufs[b], acc.at[dst_v.at[j]],
                                      ssem[b]).wait()
                if do_cnt:
                    pltpu.make_async_copy(obuf, cnt_acc.at[dst_v.at[j]],
                                          csem[b]).wait()

            wait_gather(0, 0)
            issue_scatter(0, 0)
            pltpu.async_copy(y_hbm.at[src_v.at[2]], bufs[2], gsem[2])

            def group(g, carry):
                for k in range(NB):
                    j = g * NB + 1 + k
                    b = (1 + k) % NB
                    bp = k % NB
                    wait_gather(j, b)
                    issue_scatter(j, b)
                    wait_scatter(j - 1, bp)
                    pltpu.async_copy(y_hbm.at[src_v.at[j + 2]], bufs[bp],
                                     gsem[bp])
                return carry
            lax.fori_loop(0, (NCHUNK - 4) // NB, group, 0)

            j = NCHUNK - 3
            wait_gather(j, j % NB)
            issue_scatter(j, j % NB)
            wait_scatter(j - 1, (j - 1) % NB)
            pltpu.async_copy(y_hbm.at[src_v.at[j + 2]], bufs[(j + 2) % NB],
                             gsem[(j + 2) % NB])
            for j in (NCHUNK - 2, NCHUNK - 1):
                wait_gather(j, j % NB)
                issue_scatter(j, j % NB)
                wait_scatter(j - 1, (j - 1) % NB)
            wait_scatter(NCHUNK - 1, (NCHUNK - 1) % NB)

            plsc.subcore_barrier()
            pltpu.sync_copy(acc.at[pl.ds(base, ROWS_PER_SUB)],
                            parts_out.at[cid, pl.ds(base, ROWS_PER_SUB)])
            if do_cnt:
                pltpu.sync_copy(cnt_acc.at[pl.ds(base, ROWS_PER_SUB)],
                                cnt_out.at[pl.ds(base, ROWS_PER_SUB)])

        @pl.when(cid == 0)
        def _():
            pipeline(y0_hbm, with_cnt)

        @pl.when(cid == 1)
        def _():
            pipeline(y1_hbm, False)

    return pl.kernel(body, out_type=tuple(out_type), mesh=mesh,
                     scratch_types=tuple(scratch),
                     compiler_params=pltpu.CompilerParams(
                         use_tc_tiling_on_sc=False))


_segsum_cnt = _sc_segsum(True)
_segsum = _sc_segsum(False)


_RB = 1000
_GRID = N // _RB


def _mm_body(x_ref, w_ref, o0_ref, o1_ref):
    y = jnp.dot(x_ref[...], w_ref[...], preferred_element_type=jnp.float32)
    o0_ref[...] = y[:, :HD]
    o1_ref[...] = y[:, HD:]


def _matmul(x, w):
    half = jax.ShapeDtypeStruct((N, HD), jnp.float32)
    hrow = pl.BlockSpec((_RB, HD), lambda i: (i, 0))
    return pl.pallas_call(
        _mm_body,
        grid=(_GRID,),
        in_specs=[pl.BlockSpec((_RB, D), lambda i: (i, 0)),
                  pl.BlockSpec((D, D), lambda i: (0, 0))],
        out_specs=[hrow, hrow],
        out_shape=[half, half],
    )(x, w)


def _layer_body(relu, wnext, p0, p1, c, xin, wr, b, wn,
                h_ref, y0_ref=None, y1_ref=None):
    inv = 1.0 / jnp.maximum(c[:, :1], 1.0)
    agg = jnp.concatenate([p0[...], p1[...]], axis=-1)
    h = agg * inv + b[...] + jnp.dot(
        xin[...], wr[...], preferred_element_type=jnp.float32)
    if relu:
        h = jnp.maximum(h, 0.0)
    h_ref[...] = h
    if wnext:
        y = jnp.dot(h, wn[...], preferred_element_type=jnp.float32)
        y0_ref[...] = y[:, :HD]
        y1_ref[...] = y[:, HD:]


def _layer(parts, cnt, xin, wr, b, wnext, relu):
    has_next = wnext is not None
    body = functools.partial(_layer_body, relu, has_next)
    row = pl.BlockSpec((_RB, D), lambda i: (i, 0))
    hrow = pl.BlockSpec((_RB, HD), lambda i: (i, 0))
    c_spec = pl.BlockSpec((_RB, 16), lambda i: (i, 0))
    w_spec = pl.BlockSpec((D, D), lambda i: (0, 0))
    b_spec = pl.BlockSpec((1, D), lambda i: (0, 0))
    out_shape = [jax.ShapeDtypeStruct((N, D), jnp.float32)]
    out_specs = [row]
    if has_next:
        out_shape += [jax.ShapeDtypeStruct((N, HD), jnp.float32)] * 2
        out_specs += [hrow, hrow]
    wn = wnext if has_next else jnp.zeros((D, D), jnp.float32)
    res = pl.pallas_call(
        body,
        grid=(_GRID,),
        in_specs=[hrow, hrow, c_spec, row, w_spec, b_spec, w_spec],
        out_specs=out_specs,
        out_shape=out_shape,
    )(parts[0], parts[1], cnt, xin, wr, b, wn)
    return res if has_next else res[0]


def kernel(x, edge_index, Wl1, Wr1, b1, Wl2, Wr2, b2, Wl3, Wr3, b3):
    src = edge_index[0].astype(jnp.int32)
    dst = edge_index[1].astype(jnp.int32)
    pad = E_PAD - E
    src_r = jnp.concatenate([src, jnp.zeros((pad,), jnp.int32)]).reshape(
        NS, NCHUNK, CHUNK)
    pad_dst = PAD_DST + (jnp.arange(pad, dtype=jnp.int32) % (ACC_ROWS - N))
    dst_r = jnp.concatenate([dst, pad_dst]).reshape(NS, NCHUNK, CHUNK)

    zeros = jnp.zeros((CHUNK, HD), jnp.float32)
    ones16 = jnp.ones((CHUNK, 16), jnp.float32)
    zeros16 = jnp.zeros((CHUNK, 16), jnp.float32)

    b1r = b1.reshape(1, D)
    b2r = b2.reshape(1, D)
    b3r = b3.reshape(1, D)

    y1a, y1b = _matmul(x, Wl1)
    parts1, cntp = _segsum_cnt(y1a, y1b, src_r, dst_r, zeros, ones16, zeros16)
    cnt = cntp[:N]
    h1, y2a, y2b = _layer((parts1[0, :N], parts1[1, :N]), cnt, x, Wr1, b1r,
                          Wl2, relu=True)

    parts2 = _segsum(y2a, y2b, src_r, dst_r, zeros, ones16, zeros16)[0]
    h2, y3a, y3b = _layer((parts2[0, :N], parts2[1, :N]), cnt, h1, Wr2, b2r,
                          Wl3, relu=True)

    parts3 = _segsum(y3a, y3b, src_r, dst_r, zeros, ones16, zeros16)[0]
    out = _layer((parts3[0, :N], parts3[1, :N]), cnt, h2, Wr3, b3r, None,
                 relu=False)
    return out


# speedup vs baseline: 8.8343x; 1.6839x over previous
import functools

import jax
import jax.numpy as jnp
from jax import lax
from jax.experimental import pallas as pl
from jax.experimental.pallas import tpu as pltpu
from jax.experimental.pallas import tpu_sc as plsc

N = 10000
E = 320000
D = 128
HD = D // 2

NC = 2
NS = 16

CHUNK = 128
NCHUNK = 160
EPT = CHUNK * NCHUNK
E_PAD = NS * EPT

NB = 4
IR = 6

ACC_ROWS = 10240
ROWS_PER_SUB = ACC_ROWS // NS
Y_PER_SUB = N // NS
PAD_DST = N


def _sc_segsum(with_cnt):
    mesh = plsc.VectorSubcoreMesh(
        core_axis_name="c", subcore_axis_name="s", num_cores=NC, num_subcores=NS)

    out_type = [jax.ShapeDtypeStruct((NC, ACC_ROWS, HD), jnp.float32)]
    scratch = [
        pltpu.VMEM((IR, 2, CHUNK), jnp.int32),
        pltpu.VMEM_SHARED((N, HD), jnp.float32),
        pltpu.VMEM_SHARED((ACC_ROWS, HD), jnp.float32),
    ] + [pltpu.VMEM((CHUNK, HD), jnp.float32) for _ in range(NB)] \
      + [pltpu.SemaphoreType.DMA for _ in range(2 * NB)] \
      + [pltpu.SemaphoreType.DMA for _ in range(IR)] \
      + [pltpu.SemaphoreType.DMA]
    if with_cnt:
        out_type.append(jax.ShapeDtypeStruct((ACC_ROWS, 16), jnp.float32))
        scratch += [
            pltpu.VMEM((CHUNK, 16), jnp.float32),
            pltpu.VMEM((CHUNK, 16), jnp.float32),
            pltpu.VMEM_SHARED((ACC_ROWS, 16), jnp.float32),
        ] + [pltpu.SemaphoreType.DMA for _ in range(NB)]

    def body(y0_hbm, y1_hbm, eidx_hbm, zeros_hbm, ones_hbm, zeros16_hbm,
             *refs):
        if with_cnt:
            (parts_out, cnt_out, iring, ysh, acc, *rest) = refs
            bufs = rest[:NB]
            gsem = rest[NB:2 * NB]
            ssem = rest[2 * NB:3 * NB]
            isem = rest[3 * NB:3 * NB + IR]
            stsem = rest[3 * NB + IR]
            obuf, zbuf16, cnt_acc = rest[3 * NB + IR + 1:3 * NB + IR + 4]
            csem = rest[3 * NB + IR + 4:]
        else:
            (parts_out, iring, ysh, acc, *rest) = refs
            bufs = rest[:NB]
            gsem = rest[NB:2 * NB]
            ssem = rest[2 * NB:3 * NB]
            isem = rest[3 * NB:3 * NB + IR]
            stsem = rest[3 * NB + IR]

        cid = lax.axis_index("c")
        sid = lax.axis_index("s")
        base = sid * ROWS_PER_SUB
        ybase = sid * Y_PER_SUB

        def fetch_idx(j, m):
            pltpu.async_copy(eidx_hbm.at[sid, j], iring.at[m % IR],
                             isem[m % IR])

        def pipeline(y_hbm, do_cnt):
            pltpu.async_copy(y_hbm.at[pl.ds(ybase, Y_PER_SUB)],
                             ysh.at[pl.ds(ybase, Y_PER_SUB)], stsem)
            for j in range(3):
                fetch_idx(j, j)
            pltpu.sync_copy(zeros_hbm, bufs[0])
            for k in range(ROWS_PER_SUB // CHUNK):
                pltpu.sync_copy(bufs[0], acc.at[pl.ds(base + k * CHUNK, CHUNK)])
            if do_cnt:
                pltpu.sync_copy(ones_hbm, obuf)
                pltpu.sync_copy(zeros16_hbm, zbuf16)
                for k in range(ROWS_PER_SUB // CHUNK):
                    pltpu.sync_copy(zbuf16,
                                    cnt_acc.at[pl.ds(base + k * CHUNK, CHUNK)])
            pltpu.make_async_copy(y_hbm.at[pl.ds(ybase, Y_PER_SUB)],
                                  ysh.at[pl.ds(ybase, Y_PER_SUB)],
                                  stsem).wait()
            plsc.subcore_barrier()

            def wait_idx(j, m):
                pltpu.make_async_copy(eidx_hbm.at[sid, j], iring.at[m % IR],
                                      isem[m % IR]).wait()

            def issue_gather(m):
                pltpu.async_copy(ysh.at[iring.at[m % IR, 0]], bufs[m % NB],
                                 gsem[m % NB])

            def wait_gather(m):
                pltpu.make_async_copy(ysh.at[iring.at[m % IR, 0]],
                                      bufs[m % NB], gsem[m % NB]).wait()

            def issue_scatter(m):
                pltpu.async_copy(bufs[m % NB], acc.at[iring.at[m % IR, 1]],
                                 ssem[m % NB], add=True)
                if do_cnt:
                    pltpu.async_copy(obuf, cnt_acc.at[iring.at[m % IR, 1]],
                                     csem[m % NB], add=True)

            def wait_scatter(m):
                pltpu.make_async_copy(bufs[m % NB], acc.at[iring.at[m % IR, 1]],
                                      ssem[m % NB]).wait()
                if do_cnt:
                    pltpu.make_async_copy(obuf, cnt_acc.at[iring.at[m % IR, 1]],
                                          csem[m % NB]).wait()

            def step(j, m, first=False, fetch=True, gather=True):
                if gather:
                    wait_idx(j + 2, m + 2)
                wait_gather(m)
                issue_scatter(m)
                if not first:
                    wait_scatter(m - 2)
                if gather:
                    issue_gather(m + 2)
                if fetch:
                    fetch_idx(j + 3, m + 3)

            wait_idx(0, 0)
            issue_gather(0)
            wait_idx(1, 1)
            issue_gather(1)

            step(0, 0, first=True)
            step(1, 1, first=True)
            step(2, 2)
            step(3, 3)

            def group(g, carry):
                jb = g * 12 + 4
                for k in range(12):
                    step(jb + k, 4 + k)
                return carry
            lax.fori_loop(0, (NCHUNK - 16) // 12, group, 0)

            for j in range(NCHUNK - 12, NCHUNK - 3):
                step(j, j % 12)
            j = NCHUNK - 3
            step(j, j % 12, fetch=False)
            step(NCHUNK - 2, (NCHUNK - 2) % 12, fetch=False, gather=False)
            step(NCHUNK - 1, (NCHUNK - 1) % 12, fetch=False, gather=False)
            wait_scatter((NCHUNK - 2) % 12)
            wait_scatter((NCHUNK - 1) % 12)

            plsc.subcore_barrier()
            pltpu.sync_copy(acc.at[pl.ds(base, ROWS_PER_SUB)],
                            parts_out.at[cid, pl.ds(base, ROWS_PER_SUB)])
            if do_cnt:
                pltpu.sync_copy(cnt_acc.at[pl.ds(base, ROWS_PER_SUB)],
                                cnt_out.at[pl.ds(base, ROWS_PER_SUB)])

        @pl.when(cid == 0)
        def _():
            pipeline(y0_hbm, with_cnt)

        @pl.when(cid == 1)
        def _():
            pipeline(y1_hbm, False)

    return pl.kernel(body, out_type=tuple(out_type), mesh=mesh,
                     scratch_types=tuple(scratch),
                     compiler_params=pltpu.CompilerParams(
                         use_tc_tiling_on_sc=False))


_segsum_cnt = _sc_segsum(True)
_segsum = _sc_segsum(False)


_RB = 1000
_GRID = N // _RB


def _mm_body(x_ref, w_ref, o0_ref, o1_ref):
    y = jnp.dot(x_ref[...], w_ref[...], preferred_element_type=jnp.float32)
    o0_ref[...] = y[:, :HD]
    o1_ref[...] = y[:, HD:]


def _matmul(x, w):
    half = jax.ShapeDtypeStruct((N, HD), jnp.float32)
    hrow = pl.BlockSpec((_RB, HD), lambda i: (i, 0))
    return pl.pallas_call(
        _mm_body,
        grid=(_GRID,),
        in_specs=[pl.BlockSpec((_RB, D), lambda i: (i, 0)),
                  pl.BlockSpec((D, D), lambda i: (0, 0))],
        out_specs=[hrow, hrow],
        out_shape=[half, half],
    )(x, w)


def _layer_body(relu, wnext, p0, p1, c, xin, wr, b, wn,
                h_ref, y0_ref=None, y1_ref=None):
    inv = 1.0 / jnp.maximum(c[:, :1], 1.0)
    agg = jnp.concatenate([p0[...], p1[...]], axis=-1)
    h = agg * inv + b[...] + jnp.dot(
        xin[...], wr[...], preferred_element_type=jnp.float32)
    if relu:
        h = jnp.maximum(h, 0.0)
    h_ref[...] = h
    if wnext:
        y = jnp.dot(h, wn[...], preferred_element_type=jnp.float32)
        y0_ref[...] = y[:, :HD]
        y1_ref[...] = y[:, HD:]


def _layer(parts, cnt, xin, wr, b, wnext, relu):
    has_next = wnext is not None
    body = functools.partial(_layer_body, relu, has_next)
    row = pl.BlockSpec((_RB, D), lambda i: (i, 0))
    hrow = pl.BlockSpec((_RB, HD), lambda i: (i, 0))
    c_spec = pl.BlockSpec((_RB, 16), lambda i: (i, 0))
    w_spec = pl.BlockSpec((D, D), lambda i: (0, 0))
    b_spec = pl.BlockSpec((1, D), lambda i: (0, 0))
    out_shape = [jax.ShapeDtypeStruct((N, D), jnp.float32)]
    out_specs = [row]
    if has_next:
        out_shape += [jax.ShapeDtypeStruct((N, HD), jnp.float32)] * 2
        out_specs += [hrow, hrow]
    wn = wnext if has_next else jnp.zeros((D, D), jnp.float32)
    res = pl.pallas_call(
        body,
        grid=(_GRID,),
        in_specs=[hrow, hrow, c_spec, row, w_spec, b_spec, w_spec],
        out_specs=out_specs,
        out_shape=out_shape,
    )(parts[0], parts[1], cnt, xin, wr, b, wn)
    return res if has_next else res[0]


def kernel(x, edge_index, Wl1, Wr1, b1, Wl2, Wr2, b2, Wl3, Wr3, b3):
    src = edge_index[0].astype(jnp.int32)
    dst = edge_index[1].astype(jnp.int32)
    pad = E_PAD - E
    src_r = jnp.concatenate([src, jnp.zeros((pad,), jnp.int32)]).reshape(
        NS, NCHUNK, CHUNK)
    pad_dst = PAD_DST + (jnp.arange(pad, dtype=jnp.int32) % (ACC_ROWS - N))
    dst_r = jnp.concatenate([dst, pad_dst]).reshape(NS, NCHUNK, CHUNK)
    eidx = jnp.stack([src_r, dst_r], axis=2)

    zeros = jnp.zeros((CHUNK, HD), jnp.float32)
    ones16 = jnp.ones((CHUNK, 16), jnp.float32)
    zeros16 = jnp.zeros((CHUNK, 16), jnp.float32)

    b1r = b1.reshape(1, D)
    b2r = b2.reshape(1, D)
    b3r = b3.reshape(1, D)

    y1a, y1b = _matmul(x, Wl1)
    parts1, cntp = _segsum_cnt(y1a, y1b, eidx, zeros, ones16, zeros16)
    cnt = cntp[:N]
    h1, y2a, y2b = _layer((parts1[0, :N], parts1[1, :N]), cnt, x, Wr1, b1r,
                          Wl2, relu=True)

    parts2 = _segsum(y2a, y2b, eidx, zeros, ones16, zeros16)[0]
    h2, y3a, y3b = _layer((parts2[0, :N], parts2[1, :N]), cnt, h1, Wr2, b2r,
                          Wl3, relu=True)

    parts3 = _segsum(y3a, y3b, eidx, zeros, ones16, zeros16)[0]
    out = _layer((parts3[0, :N], parts3[1, :N]), cnt, h2, Wr3, b3r, None,
                 relu=False)
    return out


# speedup vs baseline: 9.3456x; 1.0579x over previous
import functools

import jax
import jax.numpy as jnp
from jax import lax
from jax.experimental import pallas as pl
from jax.experimental.pallas import tpu as pltpu
from jax.experimental.pallas import tpu_sc as plsc

N = 10000
E = 320000
D = 128
HD = D // 2

NC = 2
NS = 16

CHUNK = 128
NCHUNK = 160
EPT = CHUNK * NCHUNK
E_PAD = NS * EPT

NB = 4
IR = 6

ACC_ROWS = 10240
ROWS_PER_SUB = ACC_ROWS // NS
Y_PER_SUB = N // NS
PAD_DST = N


def _sc_segsum(with_cnt):
    mesh = plsc.VectorSubcoreMesh(
        core_axis_name="c", subcore_axis_name="s", num_cores=NC, num_subcores=NS)

    out_type = [jax.ShapeDtypeStruct((NC, ACC_ROWS, HD), jnp.float32)]
    scratch = [
        pltpu.VMEM((IR, 2, CHUNK), jnp.int32),
        pltpu.VMEM_SHARED((N, HD), jnp.float32),
        pltpu.VMEM_SHARED((ACC_ROWS, HD), jnp.float32),
    ] + [pltpu.VMEM((CHUNK, HD), jnp.float32) for _ in range(NB)] \
      + [pltpu.SemaphoreType.DMA for _ in range(2 * NB)] \
      + [pltpu.SemaphoreType.DMA for _ in range(IR)] \
      + [pltpu.SemaphoreType.DMA]
    if with_cnt:
        out_type.append(jax.ShapeDtypeStruct((NC, ACC_ROWS, 16), jnp.float32))
        scratch += [
            pltpu.VMEM((CHUNK, 16), jnp.float32),
            pltpu.VMEM((CHUNK, 16), jnp.float32),
            pltpu.VMEM_SHARED((ACC_ROWS, 16), jnp.float32),
        ] + [pltpu.SemaphoreType.DMA for _ in range(NB)]

    def body(y0_hbm, y1_hbm, eidx_hbm, zeros_hbm, ones_hbm, zeros16_hbm,
             *refs):
        if with_cnt:
            (parts_out, cnt_out, iring, ysh, acc, *rest) = refs
            bufs = rest[:NB]
            gsem = rest[NB:2 * NB]
            ssem = rest[2 * NB:3 * NB]
            isem = rest[3 * NB:3 * NB + IR]
            stsem = rest[3 * NB + IR]
            obuf, zbuf16, cnt_acc = rest[3 * NB + IR + 1:3 * NB + IR + 4]
            csem = rest[3 * NB + IR + 4:]
        else:
            (parts_out, iring, ysh, acc, *rest) = refs
            bufs = rest[:NB]
            gsem = rest[NB:2 * NB]
            ssem = rest[2 * NB:3 * NB]
            isem = rest[3 * NB:3 * NB + IR]
            stsem = rest[3 * NB + IR]

        cid = lax.axis_index("c")
        sid = lax.axis_index("s")
        base = sid * ROWS_PER_SUB
        ybase = sid * Y_PER_SUB

        def fetch_idx(j, m):
            pltpu.async_copy(eidx_hbm.at[sid, j], iring.at[m % IR],
                             isem[m % IR])

        def cnt_pass(lo):
            ph = lo % 12

            def cwait_idx(j, m):
                pltpu.make_async_copy(eidx_hbm.at[sid, j], iring.at[m % IR],
                                      isem[m % IR]).wait()

            def cissue(m):
                pltpu.async_copy(obuf, cnt_acc.at[iring.at[m % IR, 1]],
                                 csem[m % NB], add=True)

            def cwaits(m):
                pltpu.make_async_copy(obuf, cnt_acc.at[iring.at[m % IR, 1]],
                                      csem[m % NB]).wait()

            def cstep(j, m, first=False, fetch=True):
                cwait_idx(j, m)
                cissue(m)
                if not first:
                    cwaits(m - 2)
                if fetch:
                    fetch_idx(j + 2, m + 2)

            fetch_idx(lo, ph)
            fetch_idx(lo + 1, ph + 1)
            cstep(lo, ph, first=True)
            cstep(lo + 1, ph + 1, first=True)
            cstep(lo + 2, ph + 2)
            cstep(lo + 3, ph + 3)

            def cgroup(g, carry):
                jb = lo + 4 + g * 12
                for k in range(12):
                    cstep(jb + k, ph + 4 + k)
                return carry
            lax.fori_loop(0, (NCHUNK // 2 - 8) // 12, cgroup, 0)

            for t in range(NCHUNK // 2 - 4, NCHUNK // 2):
                cstep(lo + t, ph + t, fetch=(t < NCHUNK // 2 - 2))
            cwaits(ph + NCHUNK // 2 - 2)
            cwaits(ph + NCHUNK // 2 - 1)

        def pipeline(y_hbm, do_cnt, cnt_lo):
            pltpu.async_copy(y_hbm.at[pl.ds(ybase, Y_PER_SUB)],
                             ysh.at[pl.ds(ybase, Y_PER_SUB)], stsem)
            pltpu.sync_copy(zeros_hbm, bufs[0])
            for k in range(ROWS_PER_SUB // CHUNK):
                pltpu.sync_copy(bufs[0], acc.at[pl.ds(base + k * CHUNK, CHUNK)])
            if do_cnt:
                pltpu.sync_copy(ones_hbm, obuf)
                pltpu.sync_copy(zeros16_hbm, zbuf16)
                for k in range(ROWS_PER_SUB // CHUNK):
                    pltpu.sync_copy(zbuf16,
                                    cnt_acc.at[pl.ds(base + k * CHUNK, CHUNK)])
            plsc.subcore_barrier()
            if do_cnt:
                cnt_pass(cnt_lo)
            for j in range(3):
                fetch_idx(j, j)
            pltpu.make_async_copy(y_hbm.at[pl.ds(ybase, Y_PER_SUB)],
                                  ysh.at[pl.ds(ybase, Y_PER_SUB)],
                                  stsem).wait()
            plsc.subcore_barrier()

            def wait_idx(j, m):
                pltpu.make_async_copy(eidx_hbm.at[sid, j], iring.at[m % IR],
                                      isem[m % IR]).wait()

            def issue_gather(m):
                pltpu.async_copy(ysh.at[iring.at[m % IR, 0]], bufs[m % NB],
                                 gsem[m % NB])

            def wait_gather(m):
                pltpu.make_async_copy(ysh.at[iring.at[m % IR, 0]],
                                      bufs[m % NB], gsem[m % NB]).wait()

            def issue_scatter(m):
                pltpu.async_copy(bufs[m % NB], acc.at[iring.at[m % IR, 1]],
                                 ssem[m % NB], add=True)

            def wait_scatter(m):
                pltpu.make_async_copy(bufs[m % NB], acc.at[iring.at[m % IR, 1]],
                                      ssem[m % NB]).wait()

            def step(j, m, first=False, fetch=True, gather=True):
                if gather:
                    wait_idx(j + 2, m + 2)
                wait_gather(m)
                issue_scatter(m)
                if not first:
                    wait_scatter(m - 2)
                if gather:
                    issue_gather(m + 2)
                if fetch:
                    fetch_idx(j + 3, m + 3)

            wait_idx(0, 0)
            issue_gather(0)
            wait_idx(1, 1)
            issue_gather(1)

            step(0, 0, first=True)
            step(1, 1, first=True)
            step(2, 2)
            step(3, 3)

            def group(g, carry):
                jb = g * 12 + 4
                for k in range(12):
                    step(jb + k, 4 + k)
                return carry
            lax.fori_loop(0, (NCHUNK - 16) // 12, group, 0)

            for j in range(NCHUNK - 12, NCHUNK - 3):
                step(j, j % 12)
            j = NCHUNK - 3
            step(j, j % 12, fetch=False)
            step(NCHUNK - 2, (NCHUNK - 2) % 12, fetch=False, gather=False)
            step(NCHUNK - 1, (NCHUNK - 1) % 12, fetch=False, gather=False)
            wait_scatter((NCHUNK - 2) % 12)
            wait_scatter((NCHUNK - 1) % 12)

            plsc.subcore_barrier()
            pltpu.sync_copy(acc.at[pl.ds(base, ROWS_PER_SUB)],
                            parts_out.at[cid, pl.ds(base, ROWS_PER_SUB)])
            if do_cnt:
                pltpu.sync_copy(cnt_acc.at[pl.ds(base, ROWS_PER_SUB)],
                                cnt_out.at[cid, pl.ds(base, ROWS_PER_SUB)])

        @pl.when(cid == 0)
        def _():
            pipeline(y0_hbm, with_cnt, 0)

        @pl.when(cid == 1)
        def _():
            pipeline(y1_hbm, with_cnt, NCHUNK // 2)

    return pl.kernel(body, out_type=tuple(out_type), mesh=mesh,
                     scratch_types=tuple(scratch),
                     compiler_params=pltpu.CompilerParams(
                         use_tc_tiling_on_sc=False))


_segsum_cnt = _sc_segsum(True)
_segsum = _sc_segsum(False)


_RB = 1000
_GRID = N // _RB


def _mm_body(x_ref, w_ref, o0_ref, o1_ref):
    y = jnp.dot(x_ref[...], w_ref[...], preferred_element_type=jnp.float32)
    o0_ref[...] = y[:, :HD]
    o1_ref[...] = y[:, HD:]


def _matmul(x, w):
    half = jax.ShapeDtypeStruct((N, HD), jnp.float32)
    hrow = pl.BlockSpec((_RB, HD), lambda i: (i, 0))
    return pl.pallas_call(
        _mm_body,
        grid=(_GRID,),
        in_specs=[pl.BlockSpec((_RB, D), lambda i: (i, 0)),
                  pl.BlockSpec((D, D), lambda i: (0, 0))],
        out_specs=[hrow, hrow],
        out_shape=[half, half],
    )(x, w)


def _layer_body(relu, wnext, p0, p1, c, xin, wr, b, wn,
                h_ref, y0_ref=None, y1_ref=None):
    inv = 1.0 / jnp.maximum(c[:, :1], 1.0)
    agg = jnp.concatenate([p0[...], p1[...]], axis=-1)
    h = agg * inv + b[...] + jnp.dot(
        xin[...], wr[...], preferred_element_type=jnp.float32)
    if relu:
        h = jnp.maximum(h, 0.0)
    h_ref[...] = h
    if wnext:
        y = jnp.dot(h, wn[...], preferred_element_type=jnp.float32)
        y0_ref[...] = y[:, :HD]
        y1_ref[...] = y[:, HD:]


def _layer(parts, cnt, xin, wr, b, wnext, relu):
    has_next = wnext is not None
    body = functools.partial(_layer_body, relu, has_next)
    row = pl.BlockSpec((_RB, D), lambda i: (i, 0))
    hrow = pl.BlockSpec((_RB, HD), lambda i: (i, 0))
    c_spec = pl.BlockSpec((_RB, 16), lambda i: (i, 0))
    w_spec = pl.BlockSpec((D, D), lambda i: (0, 0))
    b_spec = pl.BlockSpec((1, D), lambda i: (0, 0))
    out_shape = [jax.ShapeDtypeStruct((N, D), jnp.float32)]
    out_specs = [row]
    if has_next:
        out_shape += [jax.ShapeDtypeStruct((N, HD), jnp.float32)] * 2
        out_specs += [hrow, hrow]
    wn = wnext if has_next else jnp.zeros((D, D), jnp.float32)
    res = pl.pallas_call(
        body,
        grid=(_GRID,),
        in_specs=[hrow, hrow, c_spec, row, w_spec, b_spec, w_spec],
        out_specs=out_specs,
        out_shape=out_shape,
    )(parts[0], parts[1], cnt, xin, wr, b, wn)
    return res if has_next else res[0]


def kernel(x, edge_index, Wl1, Wr1, b1, Wl2, Wr2, b2, Wl3, Wr3, b3):
    src = edge_index[0].astype(jnp.int32)
    dst = edge_index[1].astype(jnp.int32)
    pad = E_PAD - E
    src_r = jnp.concatenate([src, jnp.zeros((pad,), jnp.int32)]).reshape(
        NS, NCHUNK, CHUNK)
    pad_dst = PAD_DST + (jnp.arange(pad, dtype=jnp.int32) % (ACC_ROWS - N))
    dst_r = jnp.concatenate([dst, pad_dst]).reshape(NS, NCHUNK, CHUNK)
    eidx = jnp.stack([src_r, dst_r], axis=2)

    zeros = jnp.zeros((CHUNK, HD), jnp.float32)
    ones16 = jnp.ones((CHUNK, 16), jnp.float32)
    zeros16 = jnp.zeros((CHUNK, 16), jnp.float32)

    b1r = b1.reshape(1, D)
    b2r = b2.reshape(1, D)
    b3r = b3.reshape(1, D)

    y1a, y1b = _matmul(x, Wl1)
    parts1, cntp = _segsum_cnt(y1a, y1b, eidx, zeros, ones16, zeros16)
    cnt = cntp[0, :N] + cntp[1, :N]
    h1, y2a, y2b = _layer((parts1[0, :N], parts1[1, :N]), cnt, x, Wr1, b1r,
                          Wl2, relu=True)

    parts2 = _segsum(y2a, y2b, eidx, zeros, ones16, zeros16)[0]
    h2, y3a, y3b = _layer((parts2[0, :N], parts2[1, :N]), cnt, h1, Wr2, b2r,
                          Wl3, relu=True)

    parts3 = _segsum(y3a, y3b, eidx, zeros, ones16, zeros16)[0]
    out = _layer((parts3[0, :N], parts3[1, :N]), cnt, h2, Wr3, b3r, None,
                 relu=False)
    return out


# speedup vs baseline: 11.2353x; 1.2022x over previous
import functools

import jax
import jax.numpy as jnp
from jax import lax
from jax.experimental import pallas as pl
from jax.experimental.pallas import tpu as pltpu
from jax.experimental.pallas import tpu_sc as plsc

N = 10000
E = 320000
D = 128
HD = D // 2

NC = 2
NS = 16

CHUNK = 128
NCHUNK = 160
EPT = CHUNK * NCHUNK
E_PAD = NS * EPT

NB = 6
IR = 6

ACC_ROWS = 10240
ROWS_PER_SUB = ACC_ROWS // NS
Y_PER_SUB = N // NS
PAD_DST = N


def _sc_segsum(with_cnt):
    mesh = plsc.VectorSubcoreMesh(
        core_axis_name="c", subcore_axis_name="s", num_cores=NC, num_subcores=NS)

    out_type = [jax.ShapeDtypeStruct((NC, ACC_ROWS, HD), jnp.bfloat16)]
    scratch = [
        pltpu.VMEM((IR, 2, CHUNK), jnp.int32),
        pltpu.VMEM_SHARED((N, HD), jnp.bfloat16),
        pltpu.VMEM_SHARED((ACC_ROWS, HD), jnp.bfloat16),
    ] + [pltpu.VMEM((CHUNK, HD), jnp.bfloat16) for _ in range(NB)] \
      + [pltpu.SemaphoreType.DMA for _ in range(2 * NB)] \
      + [pltpu.SemaphoreType.DMA for _ in range(IR)] \
      + [pltpu.SemaphoreType.DMA]
    if with_cnt:
        out_type.append(jax.ShapeDtypeStruct((NC, ACC_ROWS, 16), jnp.float32))
        scratch += [
            pltpu.VMEM((CHUNK, 16), jnp.float32),
            pltpu.VMEM((CHUNK, 16), jnp.float32),
            pltpu.VMEM_SHARED((ACC_ROWS, 16), jnp.float32),
        ] + [pltpu.SemaphoreType.DMA for _ in range(NB)]

    def body(y0_hbm, y1_hbm, eidx_hbm, zeros_hbm, ones_hbm, zeros16_hbm,
             *refs):
        if with_cnt:
            (parts_out, cnt_out, iring, ysh, acc, *rest) = refs
            bufs = rest[:NB]
            gsem = rest[NB:2 * NB]
            ssem = rest[2 * NB:3 * NB]
            isem = rest[3 * NB:3 * NB + IR]
            stsem = rest[3 * NB + IR]
            obuf, zbuf16, cnt_acc = rest[3 * NB + IR + 1:3 * NB + IR + 4]
            csem = rest[3 * NB + IR + 4:]
        else:
            (parts_out, iring, ysh, acc, *rest) = refs
            bufs = rest[:NB]
            gsem = rest[NB:2 * NB]
            ssem = rest[2 * NB:3 * NB]
            isem = rest[3 * NB:3 * NB + IR]
            stsem = rest[3 * NB + IR]

        cid = lax.axis_index("c")
        sid = lax.axis_index("s")
        base = sid * ROWS_PER_SUB
        ybase = sid * Y_PER_SUB

        def fetch_idx(j, m):
            pltpu.async_copy(eidx_hbm.at[sid, j], iring.at[m % IR],
                             isem[m % IR])

        def cnt_pass(lo):
            ph = lo % 12

            def cwait_idx(j, m):
                pltpu.make_async_copy(eidx_hbm.at[sid, j], iring.at[m % IR],
                                      isem[m % IR]).wait()

            def cissue(m):
                pltpu.async_copy(obuf, cnt_acc.at[iring.at[m % IR, 1]],
                                 csem[m % NB], add=True)

            def cwaits(m):
                pltpu.make_async_copy(obuf, cnt_acc.at[iring.at[m % IR, 1]],
                                      csem[m % NB]).wait()

            def cstep(j, m, first=False, fetch=True):
                cwait_idx(j, m)
                cissue(m)
                if not first:
                    cwaits(m - 2)
                if fetch:
                    fetch_idx(j + 2, m + 2)

            fetch_idx(lo, ph)
            fetch_idx(lo + 1, ph + 1)
            cstep(lo, ph, first=True)
            cstep(lo + 1, ph + 1, first=True)
            cstep(lo + 2, ph + 2)
            cstep(lo + 3, ph + 3)

            def cgroup(g, carry):
                jb = lo + 4 + g * 12
                for k in range(12):
                    cstep(jb + k, ph + 4 + k)
                return carry
            lax.fori_loop(0, (NCHUNK // 2 - 8) // 12, cgroup, 0)

            for t in range(NCHUNK // 2 - 4, NCHUNK // 2):
                cstep(lo + t, ph + t, fetch=(t < NCHUNK // 2 - 2))
            cwaits(ph + NCHUNK // 2 - 2)
            cwaits(ph + NCHUNK // 2 - 1)

        def pipeline(y_hbm, do_cnt, cnt_lo):
            pltpu.async_copy(y_hbm.at[pl.ds(ybase, Y_PER_SUB)],
                             ysh.at[pl.ds(ybase, Y_PER_SUB)], stsem)
            pltpu.sync_copy(zeros_hbm, bufs[0])
            for k in range(ROWS_PER_SUB // CHUNK):
                pltpu.sync_copy(bufs[0], acc.at[pl.ds(base + k * CHUNK, CHUNK)])
            if do_cnt:
                pltpu.sync_copy(ones_hbm, obuf)
                pltpu.sync_copy(zeros16_hbm, zbuf16)
                for k in range(ROWS_PER_SUB // CHUNK):
                    pltpu.sync_copy(zbuf16,
                                    cnt_acc.at[pl.ds(base + k * CHUNK, CHUNK)])
            plsc.subcore_barrier()
            if do_cnt:
                cnt_pass(cnt_lo)
            for j in range(3):
                fetch_idx(j, j)
            pltpu.make_async_copy(y_hbm.at[pl.ds(ybase, Y_PER_SUB)],
                                  ysh.at[pl.ds(ybase, Y_PER_SUB)],
                                  stsem).wait()
            plsc.subcore_barrier()

            def wait_idx(j, m):
                pltpu.make_async_copy(eidx_hbm.at[sid, j], iring.at[m % IR],
                                      isem[m % IR]).wait()

            def issue_gather(m):
                pltpu.async_copy(ysh.at[iring.at[m % IR, 0]], bufs[m % NB],
                                 gsem[m % NB])

            def wait_gather(m):
                pltpu.make_async_copy(ysh.at[iring.at[m % IR, 0]],
                                      bufs[m % NB], gsem[m % NB]).wait()

            def issue_scatter(m):
                pltpu.async_copy(bufs[m % NB], acc.at[iring.at[m % IR, 1]],
                                 ssem[m % NB], add=True)

            def wait_scatter(m):
                pltpu.make_async_copy(bufs[m % NB], acc.at[iring.at[m % IR, 1]],
                                      ssem[m % NB]).wait()

            def step(j, m, first=False, fetch=True, gather=True):
                if gather:
                    wait_idx(j + 2, m + 2)
                wait_gather(m)
                issue_scatter(m)
                if not first:
                    wait_scatter(m - 3)
                if gather:
                    issue_gather(m + 2)
                if fetch:
                    fetch_idx(j + 3, m + 3)

            wait_idx(0, 0)
            issue_gather(0)
            wait_idx(1, 1)
            issue_gather(1)

            step(0, 0, first=True)
            step(1, 1, first=True)
            step(2, 2, first=True)
            step(3, 3)

            def group(g, carry):
                jb = g * 12 + 4
                for k in range(12):
                    step(jb + k, 4 + k)
                return carry
            lax.fori_loop(0, (NCHUNK - 16) // 12, group, 0)

            for j in range(NCHUNK - 12, NCHUNK - 3):
                step(j, j % 12)
            j = NCHUNK - 3
            step(j, j % 12, fetch=False)
            step(NCHUNK - 2, (NCHUNK - 2) % 12, fetch=False, gather=False)
            step(NCHUNK - 1, (NCHUNK - 1) % 12, fetch=False, gather=False)
            wait_scatter((NCHUNK - 3) % 12)
            wait_scatter((NCHUNK - 2) % 12)
            wait_scatter((NCHUNK - 1) % 12)

            plsc.subcore_barrier()
            pltpu.sync_copy(acc.at[pl.ds(base, ROWS_PER_SUB)],
                            parts_out.at[cid, pl.ds(base, ROWS_PER_SUB)])
            if do_cnt:
                pltpu.sync_copy(cnt_acc.at[pl.ds(base, ROWS_PER_SUB)],
                                cnt_out.at[cid, pl.ds(base, ROWS_PER_SUB)])

        @pl.when(cid == 0)
        def _():
            pipeline(y0_hbm, with_cnt, 0)

        @pl.when(cid == 1)
        def _():
            pipeline(y1_hbm, with_cnt, NCHUNK // 2)

    return pl.kernel(body, out_type=tuple(out_type), mesh=mesh,
                     scratch_types=tuple(scratch),
                     compiler_params=pltpu.CompilerParams(
                         use_tc_tiling_on_sc=False))


_segsum_cnt = _sc_segsum(True)
_segsum = _sc_segsum(False)


_RB = 1000
_GRID = N // _RB


def _mm_body(x_ref, w_ref, o0_ref, o1_ref):
    y = jnp.dot(x_ref[...], w_ref[...], preferred_element_type=jnp.float32)
    o0_ref[...] = y[:, :HD].astype(jnp.bfloat16)
    o1_ref[...] = y[:, HD:].astype(jnp.bfloat16)


def _matmul(x, w):
    half = jax.ShapeDtypeStruct((N, HD), jnp.bfloat16)
    hrow = pl.BlockSpec((_RB, HD), lambda i: (i, 0))
    return pl.pallas_call(
        _mm_body,
        grid=(_GRID,),
        in_specs=[pl.BlockSpec((_RB, D), lambda i: (i, 0)),
                  pl.BlockSpec((D, D), lambda i: (0, 0))],
        out_specs=[hrow, hrow],
        out_shape=[half, half],
    )(x, w)


def _layer_body(relu, wnext, p0, p1, c, xin, wr, b, wn,
                h_ref, y0_ref=None, y1_ref=None):
    inv = 1.0 / jnp.maximum(c[:, :1], 1.0)
    agg = jnp.concatenate([p0[...], p1[...]], axis=-1).astype(jnp.float32)
    h = agg * inv + b[...] + jnp.dot(
        xin[...], wr[...], preferred_element_type=jnp.float32)
    if relu:
        h = jnp.maximum(h, 0.0)
    h_ref[...] = h
    if wnext:
        y = jnp.dot(h, wn[...], preferred_element_type=jnp.float32)
        y0_ref[...] = y[:, :HD].astype(jnp.bfloat16)
        y1_ref[...] = y[:, HD:].astype(jnp.bfloat16)


def _layer(parts, cnt, xin, wr, b, wnext, relu):
    has_next = wnext is not None
    body = functools.partial(_layer_body, relu, has_next)
    row = pl.BlockSpec((_RB, D), lambda i: (i, 0))
    hrow = pl.BlockSpec((_RB, HD), lambda i: (i, 0))
    c_spec = pl.BlockSpec((_RB, 16), lambda i: (i, 0))
    w_spec = pl.BlockSpec((D, D), lambda i: (0, 0))
    b_spec = pl.BlockSpec((1, D), lambda i: (0, 0))
    out_shape = [jax.ShapeDtypeStruct((N, D), jnp.float32)]
    out_specs = [row]
    if has_next:
        out_shape += [jax.ShapeDtypeStruct((N, HD), jnp.bfloat16)] * 2
        out_specs += [hrow, hrow]
    wn = wnext if has_next else jnp.zeros((D, D), jnp.float32)
    res = pl.pallas_call(
        body,
        grid=(_GRID,),
        in_specs=[hrow, hrow, c_spec, row, w_spec, b_spec, w_spec],
        out_specs=out_specs,
        out_shape=out_shape,
    )(parts[0], parts[1], cnt, xin, wr, b, wn)
    return res if has_next else res[0]


def kernel(x, edge_index, Wl1, Wr1, b1, Wl2, Wr2, b2, Wl3, Wr3, b3):
    src = edge_index[0].astype(jnp.int32)
    dst = edge_index[1].astype(jnp.int32)
    pad = E_PAD - E
    src_r = jnp.concatenate([src, jnp.zeros((pad,), jnp.int32)]).reshape(
        NS, NCHUNK, CHUNK)
    pad_dst = PAD_DST + (jnp.arange(pad, dtype=jnp.int32) % (ACC_ROWS - N))
    dst_r = jnp.concatenate([dst, pad_dst]).reshape(NS, NCHUNK, CHUNK)
    eidx = jnp.stack([src_r, dst_r], axis=2)

    zeros = jnp.zeros((CHUNK, HD), jnp.bfloat16)
    ones16 = jnp.ones((CHUNK, 16), jnp.float32)
    zeros16 = jnp.zeros((CHUNK, 16), jnp.float32)

    b1r = b1.reshape(1, D)
    b2r = b2.reshape(1, D)
    b3r = b3.reshape(1, D)

    y1a, y1b = _matmul(x, Wl1)
    parts1, cntp = _segsum_cnt(y1a, y1b, eidx, zeros, ones16, zeros16)
    cnt = cntp[0, :N] + cntp[1, :N]
    h1, y2a, y2b = _layer((parts1[0, :N], parts1[1, :N]), cnt, x, Wr1, b1r,
                          Wl2, relu=True)

    parts2 = _segsum(y2a, y2b, eidx, zeros, ones16, zeros16)[0]
    h2, y3a, y3b = _layer((parts2[0, :N], parts2[1, :N]), cnt, h1, Wr2, b2r,
                          Wl3, relu=True)

    parts3 = _segsum(y3a, y3b, eidx, zeros, ones16, zeros16)[0]
    out = _layer((parts3[0, :N], parts3[1, :N]), cnt, h2, Wr3, b3r, None,
                 relu=False)
    return out


# speedup vs baseline: 11.6054x; 1.0329x over previous
import functools

import jax
import jax.numpy as jnp
from jax import lax
from jax.experimental import pallas as pl
from jax.experimental.pallas import tpu as pltpu
from jax.experimental.pallas import tpu_sc as plsc

N = 10000
E = 320000
D = 128
HD = D // 2

NC = 2
NS = 16

CHUNK = 128
NCHUNK = 160
EPT = CHUNK * NCHUNK
E_PAD = NS * EPT

NB = 6
IR = 6

ACC_ROWS = 10240
ROWS_PER_SUB = ACC_ROWS // NS
Y_PER_SUB = N // NS
PAD_DST = N


def _sc_segsum(with_cnt):
    mesh = plsc.VectorSubcoreMesh(
        core_axis_name="c", subcore_axis_name="s", num_cores=NC, num_subcores=NS)

    out_type = [jax.ShapeDtypeStruct((NC, ACC_ROWS, HD), jnp.bfloat16)]
    scratch = [
        pltpu.VMEM((IR, 2, CHUNK), jnp.int32),
        pltpu.VMEM_SHARED((N, HD), jnp.bfloat16),
        pltpu.VMEM_SHARED((ACC_ROWS, HD), jnp.bfloat16),
    ] + [pltpu.VMEM((CHUNK, HD), jnp.bfloat16) for _ in range(NB)] \
      + [pltpu.SemaphoreType.DMA for _ in range(2 * NB)] \
      + [pltpu.SemaphoreType.DMA for _ in range(IR)] \
      + [pltpu.SemaphoreType.DMA]
    if with_cnt:
        out_type.append(jax.ShapeDtypeStruct((NC, ACC_ROWS, 16), jnp.float32))
        scratch += [
            pltpu.VMEM((CHUNK, 16), jnp.float32),
            pltpu.VMEM((CHUNK, 16), jnp.float32),
            pltpu.VMEM_SHARED((ACC_ROWS, 16), jnp.float32),
        ] + [pltpu.SemaphoreType.DMA for _ in range(NB)]

    def body(y0_hbm, y1_hbm, eidx_hbm, zeros_hbm, ones_hbm, zeros16_hbm,
             *refs):
        if with_cnt:
            (parts_out, cnt_out, iring, ysh, acc, *rest) = refs
            bufs = rest[:NB]
            gsem = rest[NB:2 * NB]
            ssem = rest[2 * NB:3 * NB]
            isem = rest[3 * NB:3 * NB + IR]
            stsem = rest[3 * NB + IR]
            obuf, zbuf16, cnt_acc = rest[3 * NB + IR + 1:3 * NB + IR + 4]
            csem = rest[3 * NB + IR + 4:]
        else:
            (parts_out, iring, ysh, acc, *rest) = refs
            bufs = rest[:NB]
            gsem = rest[NB:2 * NB]
            ssem = rest[2 * NB:3 * NB]
            isem = rest[3 * NB:3 * NB + IR]
            stsem = rest[3 * NB + IR]

        cid = lax.axis_index("c")
        sid = lax.axis_index("s")
        base = sid * ROWS_PER_SUB
        ybase = sid * Y_PER_SUB

        def fetch_idx(j, m):
            pltpu.async_copy(eidx_hbm.at[sid, j], iring.at[m % IR],
                             isem[m % IR])

        def pipeline(y_hbm, do_cnt, clo, chi):
            pltpu.async_copy(y_hbm.at[pl.ds(ybase, Y_PER_SUB)],
                             ysh.at[pl.ds(ybase, Y_PER_SUB)], stsem)
            pltpu.sync_copy(zeros_hbm, bufs[0])
            for k in range(ROWS_PER_SUB // CHUNK):
                pltpu.sync_copy(bufs[0], acc.at[pl.ds(base + k * CHUNK, CHUNK)])
            if do_cnt:
                pltpu.sync_copy(ones_hbm, obuf)
                pltpu.sync_copy(zeros16_hbm, zbuf16)
                for k in range(ROWS_PER_SUB // CHUNK):
                    pltpu.sync_copy(zbuf16,
                                    cnt_acc.at[pl.ds(base + k * CHUNK, CHUNK)])
            for j in range(3):
                fetch_idx(j, j)
            pltpu.make_async_copy(y_hbm.at[pl.ds(ybase, Y_PER_SUB)],
                                  ysh.at[pl.ds(ybase, Y_PER_SUB)],
                                  stsem).wait()
            plsc.subcore_barrier()

            def wait_idx(j, m):
                pltpu.make_async_copy(eidx_hbm.at[sid, j], iring.at[m % IR],
                                      isem[m % IR]).wait()

            def issue_gather(m):
                pltpu.async_copy(ysh.at[iring.at[m % IR, 0]], bufs[m % NB],
                                 gsem[m % NB])

            def wait_gather(m):
                pltpu.make_async_copy(ysh.at[iring.at[m % IR, 0]],
                                      bufs[m % NB], gsem[m % NB]).wait()

            def issue_scatter(m, icnt):
                pltpu.async_copy(bufs[m % NB], acc.at[iring.at[m % IR, 1]],
                                 ssem[m % NB], add=True)
                if icnt:
                    pltpu.async_copy(obuf, cnt_acc.at[iring.at[m % IR, 1]],
                                     csem[m % NB], add=True)

            def wait_scatter(m, wcnt):
                pltpu.make_async_copy(bufs[m % NB], acc.at[iring.at[m % IR, 1]],
                                      ssem[m % NB]).wait()
                if wcnt:
                    pltpu.make_async_copy(obuf, cnt_acc.at[iring.at[m % IR, 1]],
                                          csem[m % NB]).wait()

            def step(j, m, jp, first=False, fetch=True, gather=True):
                icnt = do_cnt and clo <= jp < chi
                wcnt = do_cnt and clo <= jp - 3 < chi
                if gather:
                    wait_idx(j + 2, m + 2)
                wait_gather(m)
                issue_scatter(m, icnt)
                if not first:
                    wait_scatter(m - 3, wcnt)
                if gather:
                    issue_gather(m + 2)
                if fetch:
                    fetch_idx(j + 3, m + 3)

            wait_idx(0, 0)
            issue_gather(0)
            wait_idx(1, 1)
            issue_gather(1)

            step(0, 0, 0, first=True)
            step(1, 1, 1, first=True)
            step(2, 2, 2, first=True)
            step(3, 3, 3)

            def group_a(g, carry):
                jb = g * 12 + 4
                for k in range(12):
                    step(jb + k, 4 + k, 4 + k)
                return carry
            lax.fori_loop(0, 6, group_a, 0)

            for j in range(76, 80):
                step(j, j % 12, j)

            def group_b(g, carry):
                jb = 80 + g * 12
                for k in range(12):
                    step(jb + k, (8 + k) % 12, 80 + k)
                return carry
            lax.fori_loop(0, 5, group_b, 0)

            for j in range(140, NCHUNK - 3):
                step(j, j % 12, j)
            j = NCHUNK - 3
            step(j, j % 12, j, fetch=False)
            step(NCHUNK - 2, (NCHUNK - 2) % 12, NCHUNK - 2,
                 fetch=False, gather=False)
            step(NCHUNK - 1, (NCHUNK - 1) % 12, NCHUNK - 1,
                 fetch=False, gather=False)
            for j in (NCHUNK - 3, NCHUNK - 2, NCHUNK - 1):
                wait_scatter(j % 12, do_cnt and clo <= j < chi)

            plsc.subcore_barrier()
            pltpu.sync_copy(acc.at[pl.ds(base, ROWS_PER_SUB)],
                            parts_out.at[cid, pl.ds(base, ROWS_PER_SUB)])
            if do_cnt:
                pltpu.sync_copy(cnt_acc.at[pl.ds(base, ROWS_PER_SUB)],
                                cnt_out.at[cid, pl.ds(base, ROWS_PER_SUB)])

        @pl.when(cid == 0)
        def _():
            pipeline(y0_hbm, with_cnt, 0, 76)

        @pl.when(cid == 1)
        def _():
            pipeline(y1_hbm, with_cnt, 76, NCHUNK)

    return pl.kernel(body, out_type=tuple(out_type), mesh=mesh,
                     scratch_types=tuple(scratch),
                     compiler_params=pltpu.CompilerParams(
                         use_tc_tiling_on_sc=False))


_segsum_cnt = _sc_segsum(True)
_segsum = _sc_segsum(False)


_RB = 2000
_GRID = N // _RB


def _mm_body(x_ref, w_ref, o0_ref, o1_ref):
    y = jnp.dot(x_ref[...], w_ref[...], preferred_element_type=jnp.float32)
    o0_ref[...] = y[:, :HD].astype(jnp.bfloat16)
    o1_ref[...] = y[:, HD:].astype(jnp.bfloat16)


def _matmul(x, w):
    half = jax.ShapeDtypeStruct((N, HD), jnp.bfloat16)
    hrow = pl.BlockSpec((_RB, HD), lambda i: (i, 0))
    return pl.pallas_call(
        _mm_body,
        grid=(_GRID,),
        in_specs=[pl.BlockSpec((_RB, D), lambda i: (i, 0)),
                  pl.BlockSpec((D, D), lambda i: (0, 0))],
        out_specs=[hrow, hrow],
        out_shape=[half, half],
    )(x, w)


def _layer_body(relu, wnext, p0, p1, c, xin, wr, b, wn,
                h_ref, y0_ref=None, y1_ref=None):
    inv = 1.0 / jnp.maximum(c[:, :1], 1.0)
    agg = jnp.concatenate([p0[...], p1[...]], axis=-1).astype(jnp.float32)
    h = agg * inv + b[...] + jnp.dot(
        xin[...], wr[...], preferred_element_type=jnp.float32)
    if relu:
        h = jnp.maximum(h, 0.0)
    h_ref[...] = h
    if wnext:
        y = jnp.dot(h, wn[...], preferred_element_type=jnp.float32)
        y0_ref[...] = y[:, :HD].astype(jnp.bfloat16)
        y1_ref[...] = y[:, HD:].astype(jnp.bfloat16)


def _layer(parts, cnt, xin, wr, b, wnext, relu):
    has_next = wnext is not None
    body = functools.partial(_layer_body, relu, has_next)
    row = pl.BlockSpec((_RB, D), lambda i: (i, 0))
    hrow = pl.BlockSpec((_RB, HD), lambda i: (i, 0))
    c_spec = pl.BlockSpec((_RB, 16), lambda i: (i, 0))
    w_spec = pl.BlockSpec((D, D), lambda i: (0, 0))
    b_spec = pl.BlockSpec((1, D), lambda i: (0, 0))
    out_shape = [jax.ShapeDtypeStruct((N, D), jnp.float32)]
    out_specs = [row]
    if has_next:
        out_shape += [jax.ShapeDtypeStruct((N, HD), jnp.bfloat16)] * 2
        out_specs += [hrow, hrow]
    wn = wnext if has_next else jnp.zeros((D, D), jnp.float32)
    res = pl.pallas_call(
        body,
        grid=(_GRID,),
        in_specs=[hrow, hrow, c_spec, row, w_spec, b_spec, w_spec],
        out_specs=out_specs,
        out_shape=out_shape,
    )(parts[0], parts[1], cnt, xin, wr, b, wn)
    return res if has_next else res[0]


def kernel(x, edge_index, Wl1, Wr1, b1, Wl2, Wr2, b2, Wl3, Wr3, b3):
    src = edge_index[0].astype(jnp.int32)
    dst = edge_index[1].astype(jnp.int32)
    pad = E_PAD - E
    src_r = jnp.concatenate([src, jnp.zeros((pad,), jnp.int32)]).reshape(
        NS, NCHUNK, CHUNK)
    pad_dst = PAD_DST + (jnp.arange(pad, dtype=jnp.int32) % (ACC_ROWS - N))
    dst_r = jnp.concatenate([dst, pad_dst]).reshape(NS, NCHUNK, CHUNK)
    eidx = jnp.stack([src_r, dst_r], axis=2)

    zeros = jnp.zeros((CHUNK, HD), jnp.bfloat16)
    ones16 = jnp.ones((CHUNK, 16), jnp.float32)
    zeros16 = jnp.zeros((CHUNK, 16), jnp.float32)

    b1r = b1.reshape(1, D)
    b2r = b2.reshape(1, D)
    b3r = b3.reshape(1, D)

    y1a, y1b = _matmul(x, Wl1)
    parts1, cntp = _segsum_cnt(y1a, y1b, eidx, zeros, ones16, zeros16)
    cnt = cntp[0, :N] + cntp[1, :N]
    h1, y2a, y2b = _layer((parts1[0, :N], parts1[1, :N]), cnt, x, Wr1, b1r,
                          Wl2, relu=True)

    parts2 = _segsum(y2a, y2b, eidx, zeros, ones16, zeros16)[0]
    h2, y3a, y3b = _layer((parts2[0, :N], parts2[1, :N]), cnt, h1, Wr2, b2r,
                          Wl3, relu=True)

    parts3 = _segsum(y3a, y3b, eidx, zeros, ones16, zeros16)[0]
    out = _layer((parts3[0, :N], parts3[1, :N]), cnt, h2, Wr3, b3r, None,
                 relu=False)
    return out
